# baseline (speedup 1.0000x reference)
import functools

import jax
import jax.numpy as jnp
from jax import lax
from jax.experimental import pallas as pl
from jax.experimental.pallas import tpu as pltpu
from jax.experimental.pallas import tpu_sc as plsc

N = 10000
E = 320000
CH = 128
NW = 32
EW = 10240
E_PAD = NW * EW
K_CH = EW // CH
NBUF = 2
SEG = 40
K_PAIR = 2 * K_CH
K_FAST = 120
NSEG_FAST = K_FAST // SEG
NSEG_MAX = NSEG_FAST
C_FAST = 0
K_OUT = K_CH // NBUF
N_ACC = 10112
R_T = N_ACC // 16
F_DEG = 128


def _sc_mesh():
    return plsc.VectorSubcoreMesh(core_axis_name="c", subcore_axis_name="s")


def _sc_scatter_rows(u, rowi2, coli2, zeros, f):
    @functools.partial(
        pl.kernel,
        out_type=jax.ShapeDtypeStruct((2 * N_ACC, f), jnp.float32),
        mesh=_sc_mesh(),
        scratch_types=[
            pltpu.VMEM((SEG, CH), jnp.int32),
            pltpu.VMEM((SEG, CH), jnp.int32),
            pltpu.VMEM((NBUF, CH, f), jnp.float32),
            pltpu.VMEM_SHARED((N_ACC, f), jnp.float32),
            pltpu.SemaphoreType.DMA((NBUF,)),
            pltpu.SemaphoreType.DMA((NBUF,)),
        ],
    )
    def k(u_hbm, row_hbm, col_hbm, z_hbm, out_hbm,
          rowbuf, colbuf, rows, acc, gsem, ssem):
        c = lax.axis_index("c")
        s = lax.axis_index("s")
        base = jnp.where(c == C_FAST, 0, K_FAST)
        pltpu.sync_copy(z_hbm.at[pl.ds(s * R_T, R_T)], acc.at[pl.ds(s * R_T, R_T)])
        plsc.subcore_barrier()

        def seg_work(off):
            pltpu.sync_copy(row_hbm.at[s, pl.ds(off, SEG)], rowbuf)
            pltpu.sync_copy(col_hbm.at[s, pl.ds(off, SEG)], colbuf)
            for b in range(NBUF):
                pltpu.async_copy(u_hbm.at[rowbuf.at[b]], rows.at[b], gsem.at[b])

            def body(g, carry2):
                for b in range(NBUF):
                    j = g * NBUF + b
                    bp = (b - 1) % NBUF
                    pltpu.make_async_copy(
                        u_hbm.at[rowbuf.at[j]], rows.at[b], gsem.at[b]).wait()
                    pltpu.async_copy(
                        rows.at[b], acc.at[colbuf.at[j]], ssem.at[b], add=True)
                    jp = j - 1

                    @pl.when(jnp.logical_and(jp >= 0, jp + NBUF < SEG))
                    def _():
                        pltpu.make_async_copy(
                            rows.at[bp], acc.at[colbuf.at[jp]],
                            ssem.at[bp]).wait()
                        pltpu.async_copy(
                            u_hbm.at[rowbuf.at[jp + NBUF]], rows.at[bp],
                            gsem.at[bp])
                return carry2

            lax.fori_loop(0, SEG // NBUF, body, 0)
            for b in range(NBUF):
                j = SEG - NBUF + b
                pltpu.make_async_copy(
                    rows.at[b], acc.at[colbuf.at[j]], ssem.at[b]).wait()

        seg_work(base)
        for sg in range(1, NSEG_MAX):

            @pl.when(c == C_FAST)
            def _(sg=sg):
                seg_work(sg * SEG)

        plsc.subcore_barrier()
        pltpu.sync_copy(
            acc.at[pl.ds(s * R_T, R_T)],
            out_hbm.at[pl.ds(c * N_ACC + s * R_T, R_T)],
        )

    return k(u, rowi2, coli2, zeros)


def _sc_degree(coli3, ones, zeros):
    @functools.partial(
        pl.kernel,
        out_type=jax.ShapeDtypeStruct((2 * N_ACC, F_DEG), jnp.float32),
        mesh=_sc_mesh(),
        scratch_types=[
            pltpu.VMEM((K_CH, CH), jnp.int32),
            pltpu.VMEM((CH, F_DEG), jnp.float32),
            pltpu.VMEM_SHARED((N_ACC, F_DEG), jnp.float32),
            pltpu.SemaphoreType.DMA((NBUF,)),
        ],
    )
    def k(col_hbm, ones_hbm, z_hbm, out_hbm, colbuf, ones_v, acc, ssem):
        c = lax.axis_index("c")
        s = lax.axis_index("s")
        wid = s * 2 + c
        pltpu.sync_copy(col_hbm.at[wid], colbuf)
        pltpu.sync_copy(ones_hbm, ones_v)
        pltpu.sync_copy(z_hbm.at[pl.ds(s * R_T, R_T)], acc.at[pl.ds(s * R_T, R_T)])
        plsc.subcore_barrier()

        for b in range(NBUF):
            pltpu.async_copy(ones_v, acc.at[colbuf.at[b]], ssem.at[b], add=True)

        def body(g, carry):
            for b in range(NBUF):
                kk = g * NBUF + b
                pltpu.make_async_copy(
                    ones_v, acc.at[colbuf.at[kk - NBUF]], ssem.at[b]).wait()
                pltpu.async_copy(
                    ones_v, acc.at[colbuf.at[kk]], ssem.at[b], add=True)
            return carry

        lax.fori_loop(1, K_OUT, body, 0)
        for b in range(NBUF):
            kk = (K_OUT - 1) * NBUF + b
            pltpu.make_async_copy(
                ones_v, acc.at[colbuf.at[kk]], ssem.at[b]).wait()
        plsc.subcore_barrier()
        pltpu.sync_copy(
            acc.at[pl.ds(s * R_T, R_T)],
            out_hbm.at[pl.ds(c * N_ACC + s * R_T, R_T)],
        )

    return k(coli3, ones, zeros)


def _dinv(deg_ref):
    deg = deg_ref[0:N, 0:1] + deg_ref[N_ACC:N_ACC + N, 0:1] + 1.0
    return lax.rsqrt(deg)


def _tc_matmul(x, w1):
    def body(x_ref, w_ref, out_ref):
        out_ref[...] = jnp.dot(
            x_ref[...], w_ref[...], preferred_element_type=jnp.float32)

    return pl.pallas_call(
        body,
        out_shape=jax.ShapeDtypeStruct((N, w1.shape[1]), jnp.float32),
    )(x, w1)


def _tc_scale(h, deg):
    def body(h_ref, d_ref, out_ref):
        out_ref[...] = h_ref[...] * _dinv(d_ref)

    return pl.pallas_call(
        body,
        out_shape=jax.ShapeDtypeStruct(h.shape, jnp.float32),
    )(h, deg)


def _tc_mid(agg, u, deg, b, g, be, w_next):
    def body(a_ref, u_ref, d_ref, b_ref, g_ref, be_ref, w_ref, out_ref):
        dinv = _dinv(d_ref)
        conv = (a_ref[0:N] + a_ref[N_ACC:N_ACC + N] + u_ref[...]) * dinv + b_ref[...]
        mean = jnp.mean(conv, axis=0, keepdims=True)
        var = jnp.mean((conv - mean) ** 2, axis=0, keepdims=True)
        h = (conv - mean) * lax.rsqrt(var + 1e-5) * g_ref[...] + be_ref[...]
        h = jnp.maximum(h, 0.0)
        out_ref[...] = (
            jnp.dot(h, w_ref[...], preferred_element_type=jnp.float32) * dinv
        )

    return pl.pallas_call(
        body,
        out_shape=jax.ShapeDtypeStruct((N, w_next.shape[1]), jnp.float32),
    )(agg, u, deg, b, g, be, w_next)


def _tc_last(agg, u, deg, b, f_out):
    def body(a_ref, u_ref, d_ref, b_ref, out_ref):
        dinv = _dinv(d_ref)
        out_ref[...] = ((
            a_ref[0:N, 0:f_out] + a_ref[N_ACC:N_ACC + N, 0:f_out]
            + u_ref[0:N, 0:f_out]
        ) * dinv + b_ref[...])

    return pl.pallas_call(
        body,
        out_shape=jax.ShapeDtypeStruct((N, f_out), jnp.float32),
    )(agg, u, deg, b)


def kernel(x, edge_idx, W1, b1, g1, be1, W2, b2, g2, be2, W3, b3):
    row = edge_idx[0]
    col = edge_idx[1]
    pad = E_PAD - E
    rowp = jnp.concatenate([row, jnp.zeros((pad,), jnp.int32)])
    colp = jnp.concatenate([col, jnp.full((pad,), N, jnp.int32)])
    rowi2 = rowp.reshape(16, K_PAIR, CH)
    coli2 = colp.reshape(16, K_PAIR, CH)
    coli3 = colp.reshape(NW, K_CH, CH)

    zeros128 = jnp.zeros((N_ACC, 128), jnp.float32)
    ones128 = jnp.ones((CH, F_DEG), jnp.float32)
    f_out = W3.shape[1]
    W3p = jnp.pad(W3, ((0, 0), (0, 128 - f_out)))

    h1 = _tc_matmul(x, W1)
    deg = _sc_degree(coli3, ones128, zeros128)
    u1 = _tc_scale(h1, deg)
    agg1 = _sc_scatter_rows(u1, rowi2, coli2, zeros128, 128)
    u2 = _tc_mid(agg1, u1, deg, b1.reshape(1, -1), g1.reshape(1, -1),
                 be1.reshape(1, -1), W2)
    agg2 = _sc_scatter_rows(u2, rowi2, coli2, zeros128, 128)
    u3 = _tc_mid(agg2, u2, deg, b2.reshape(1, -1), g2.reshape(1, -1),
                 be2.reshape(1, -1), W3p)
    agg3 = _sc_scatter_rows(u3, rowi2, coli2, zeros128, 128)
    out = _tc_last(agg3, u3, deg, b3.reshape(1, -1), f_out)
    return out


# speedup vs baseline: 1.0195x; 1.0195x over previous
import functools

import jax
import jax.numpy as jnp
from jax import lax
from jax.experimental import pallas as pl
from jax.experimental.pallas import tpu as pltpu
from jax.experimental.pallas import tpu_sc as plsc

N = 10000
E = 320000
CH = 128
NW = 32
EW = 10240
E_PAD = NW * EW
K_CH = EW // CH
NBUF = 2
SEG = 40
K_PAIR = 2 * K_CH
K_FAST = 120
NSEG_FAST = K_FAST // SEG
NSEG_MAX = NSEG_FAST
C_FAST = 0
K_OUT = K_CH // NBUF
N_ACC = 10112
R_T = N_ACC // 16
F_DEG = 128


def _sc_mesh():
    return plsc.VectorSubcoreMesh(core_axis_name="c", subcore_axis_name="s")


def _sc_scatter_rows(u, rowi2, coli2, zeros, f):
    @functools.partial(
        pl.kernel,
        out_type=jax.ShapeDtypeStruct((2 * N_ACC, f), jnp.float32),
        mesh=_sc_mesh(),
        scratch_types=[
            pltpu.VMEM((SEG, CH), jnp.int32),
            pltpu.VMEM((SEG, CH), jnp.int32),
            pltpu.VMEM((NBUF, CH, f), jnp.float32),
            pltpu.VMEM_SHARED((N_ACC, f), jnp.float32),
            pltpu.SemaphoreType.DMA((NBUF,)),
            pltpu.SemaphoreType.DMA((NBUF,)),
        ],
    )
    def k(u_hbm, row_hbm, col_hbm, z_hbm, out_hbm,
          rowbuf, colbuf, rows, acc, gsem, ssem):
        c = lax.axis_index("c")
        s = lax.axis_index("s")
        base = jnp.where(c == C_FAST, 0, K_FAST)
        pltpu.sync_copy(z_hbm.at[pl.ds(s * R_T, R_T)], acc.at[pl.ds(s * R_T, R_T)])
        plsc.subcore_barrier()

        def seg_work(off):
            pltpu.sync_copy(row_hbm.at[s, pl.ds(off, SEG)], rowbuf)
            pltpu.sync_copy(col_hbm.at[s, pl.ds(off, SEG)], colbuf)
            for b in range(NBUF):
                pltpu.async_copy(u_hbm.at[rowbuf.at[b]], rows.at[b], gsem.at[b])

            def body(g, carry2):
                for b in range(NBUF):
                    j = g * NBUF + b
                    bp = (b - 1) % NBUF
                    pltpu.make_async_copy(
                        u_hbm.at[rowbuf.at[j]], rows.at[b], gsem.at[b]).wait()
                    pltpu.async_copy(
                        rows.at[b], acc.at[colbuf.at[j]], ssem.at[b], add=True)
                    jp = j - 1

                    @pl.when(jnp.logical_and(jp >= 0, jp + NBUF < SEG))
                    def _():
                        pltpu.make_async_copy(
                            rows.at[bp], acc.at[colbuf.at[jp]],
                            ssem.at[bp]).wait()
                        pltpu.async_copy(
                            u_hbm.at[rowbuf.at[jp + NBUF]], rows.at[bp],
                            gsem.at[bp])
                return carry2

            lax.fori_loop(0, SEG // NBUF, body, 0)
            for b in range(NBUF):
                j = SEG - NBUF + b
                pltpu.make_async_copy(
                    rows.at[b], acc.at[colbuf.at[j]], ssem.at[b]).wait()

        seg_work(base)
        for sg in range(1, NSEG_MAX):

            @pl.when(c == C_FAST)
            def _(sg=sg):
                seg_work(sg * SEG)

        plsc.subcore_barrier()
        pltpu.sync_copy(
            acc.at[pl.ds(s * R_T, R_T)],
            out_hbm.at[pl.ds(c * N_ACC + s * R_T, R_T)],
        )

    return k(u, rowi2, coli2, zeros)


def _sc_degree(coli3, ones, zeros):
    @functools.partial(
        pl.kernel,
        out_type=jax.ShapeDtypeStruct((2 * N_ACC, F_DEG), jnp.float32),
        mesh=_sc_mesh(),
        scratch_types=[
            pltpu.VMEM((K_CH, CH), jnp.int32),
            pltpu.VMEM((CH, F_DEG), jnp.float32),
            pltpu.VMEM_SHARED((N_ACC, F_DEG), jnp.float32),
            pltpu.SemaphoreType.DMA((NBUF,)),
        ],
    )
    def k(col_hbm, ones_hbm, z_hbm, out_hbm, colbuf, ones_v, acc, ssem):
        c = lax.axis_index("c")
        s = lax.axis_index("s")
        wid = s * 2 + c
        pltpu.sync_copy(col_hbm.at[wid], colbuf)
        pltpu.sync_copy(ones_hbm, ones_v)
        pltpu.sync_copy(z_hbm.at[pl.ds(s * R_T, R_T)], acc.at[pl.ds(s * R_T, R_T)])
        plsc.subcore_barrier()

        for b in range(NBUF):
            pltpu.async_copy(ones_v, acc.at[colbuf.at[b]], ssem.at[b], add=True)

        def body(g, carry):
            for b in range(NBUF):
                kk = g * NBUF + b
                pltpu.make_async_copy(
                    ones_v, acc.at[colbuf.at[kk - NBUF]], ssem.at[b]).wait()
                pltpu.async_copy(
                    ones_v, acc.at[colbuf.at[kk]], ssem.at[b], add=True)
            return carry

        lax.fori_loop(1, K_OUT, body, 0)
        for b in range(NBUF):
            kk = (K_OUT - 1) * NBUF + b
            pltpu.make_async_copy(
                ones_v, acc.at[colbuf.at[kk]], ssem.at[b]).wait()
        plsc.subcore_barrier()
        pltpu.sync_copy(
            acc.at[pl.ds(s * R_T, R_T)],
            out_hbm.at[pl.ds(c * N_ACC + s * R_T, R_T)],
        )

    return k(coli3, ones, zeros)


def _dinv(deg_ref):
    deg = deg_ref[0:N, 0:1] + deg_ref[N_ACC:N_ACC + N, 0:1] + 1.0
    return lax.rsqrt(deg)


def _tc_first(x, w1, deg):
    def body(x_ref, w_ref, d_ref, out_ref):
        dinv = _dinv(d_ref)
        h = jnp.dot(x_ref[...], w_ref[...], preferred_element_type=jnp.float32)
        out_ref[...] = h * dinv

    return pl.pallas_call(
        body,
        out_shape=jax.ShapeDtypeStruct((N, w1.shape[1]), jnp.float32),
    )(x, w1, deg)


def _tc_mid(agg, u, deg, b, g, be, w_next):
    def body(a_ref, u_ref, d_ref, b_ref, g_ref, be_ref, w_ref, out_ref):
        dinv = _dinv(d_ref)
        conv = (a_ref[0:N] + a_ref[N_ACC:N_ACC + N] + u_ref[...]) * dinv + b_ref[...]
        mean = jnp.mean(conv, axis=0, keepdims=True)
        var = jnp.mean((conv - mean) ** 2, axis=0, keepdims=True)
        h = (conv - mean) * lax.rsqrt(var + 1e-5) * g_ref[...] + be_ref[...]
        h = jnp.maximum(h, 0.0)
        out_ref[...] = (
            jnp.dot(h, w_ref[...], preferred_element_type=jnp.float32) * dinv
        )

    return pl.pallas_call(
        body,
        out_shape=jax.ShapeDtypeStruct((N, w_next.shape[1]), jnp.float32),
    )(agg, u, deg, b, g, be, w_next)


def _tc_last(agg, u, deg, b, f_out):
    def body(a_ref, u_ref, d_ref, b_ref, out_ref):
        dinv = _dinv(d_ref)
        out_ref[...] = ((
            a_ref[0:N, 0:f_out] + a_ref[N_ACC:N_ACC + N, 0:f_out]
            + u_ref[0:N, 0:f_out]
        ) * dinv + b_ref[...])

    return pl.pallas_call(
        body,
        out_shape=jax.ShapeDtypeStruct((N, f_out), jnp.float32),
    )(agg, u, deg, b)


def kernel(x, edge_idx, W1, b1, g1, be1, W2, b2, g2, be2, W3, b3):
    row = edge_idx[0]
    col = edge_idx[1]
    pad = E_PAD - E
    rowp = jnp.concatenate([row, jnp.zeros((pad,), jnp.int32)])
    colp = jnp.concatenate([col, jnp.full((pad,), N, jnp.int32)])
    rowi2 = rowp.reshape(16, K_PAIR, CH)
    coli2 = colp.reshape(16, K_PAIR, CH)
    coli3 = colp.reshape(NW, K_CH, CH)

    zeros128 = jnp.zeros((N_ACC, 128), jnp.float32)
    ones128 = jnp.ones((CH, F_DEG), jnp.float32)
    f_out = W3.shape[1]
    W3p = jnp.pad(W3, ((0, 0), (0, 128 - f_out)))

    deg = _sc_degree(coli3, ones128, zeros128)

    u1 = _tc_first(x, W1, deg)
    agg1 = _sc_scatter_rows(u1, rowi2, coli2, zeros128, 128)
    u2 = _tc_mid(agg1, u1, deg, b1.reshape(1, -1), g1.reshape(1, -1),
                 be1.reshape(1, -1), W2)
    agg2 = _sc_scatter_rows(u2, rowi2, coli2, zeros128, 128)
    u3 = _tc_mid(agg2, u2, deg, b2.reshape(1, -1), g2.reshape(1, -1),
                 be2.reshape(1, -1), W3p)
    agg3 = _sc_scatter_rows(u3, rowi2, coli2, zeros128, 128)
    out = _tc_last(agg3, u3, deg, b3.reshape(1, -1), f_out)
    return out
